# Initial kernel scaffold; baseline (speedup 1.0000x reference)
import jax
import jax.numpy as jnp
from jax.experimental import pallas as pl


def kernel(seq, embed, W1, b1, W2, b2, gamma, beta, Wk, Wr, br, Wo, bo):
    raise NotImplementedError("write your pallas kernel here")


# speedup vs baseline: 6.1632x; 6.1632x over previous
import jax
import jax.numpy as jnp
import numpy as np
from jax.experimental import pallas as pl
from jax.experimental.pallas import tpu as pltpu

B, L, H, V = 64, 512, 128, 32000
EMA = float(np.float32(1.0 - 0.95))
VT2 = float(np.float32(0.1 * 0.1))

TM = 256
NT = (B * L) // TM
NT_HALF = NT // 2


def _ffn_kernel(seq_ref, embed_ref, W1_ref, b1_ref, W2_ref, b2_ref,
                gam_ref, bet_ref, Wk_ref, out_ref, gbuf):
    i = pl.program_id(0)
    j = pl.program_id(1)
    base = (i * NT_HALF + j) * TM
    for mi in range(TM):
        tok = seq_ref[base + mi]
        gbuf[mi] = embed_ref[tok]
    h = gbuf[...]
    z = jnp.maximum(
        jnp.dot(h, W1_ref[...], preferred_element_type=jnp.float32)
        + b1_ref[...], 0.0)
    h = h + jnp.dot(z, W2_ref[...], preferred_element_type=jnp.float32) \
        + b2_ref[...]
    mu = jnp.mean(h, axis=-1, keepdims=True)
    xc = h - mu
    var = jnp.mean(xc * xc, axis=-1, keepdims=True)
    hn = xc * jax.lax.rsqrt(var + 1e-5) * gam_ref[...] + bet_ref[...]
    out_ref[...] = jnp.dot(hn, Wk_ref[...], preferred_element_type=jnp.float32)


def _ffn_call(seq_flat, embed, W1, b1, W2, b2, gamma, beta, Wk):
    return pl.pallas_call(
        _ffn_kernel,
        out_shape=jax.ShapeDtypeStruct((B * L, H), jnp.float32),
        grid=(2, NT_HALF),
        in_specs=[
            pl.BlockSpec(memory_space=pltpu.SMEM),
            pl.BlockSpec((V, H), lambda i, j: (0, 0)),
            pl.BlockSpec((H, 2 * H), lambda i, j: (0, 0)),
            pl.BlockSpec((1, 2 * H), lambda i, j: (0, 0)),
            pl.BlockSpec((2 * H, H), lambda i, j: (0, 0)),
            pl.BlockSpec((1, H), lambda i, j: (0, 0)),
            pl.BlockSpec((1, H), lambda i, j: (0, 0)),
            pl.BlockSpec((1, H), lambda i, j: (0, 0)),
            pl.BlockSpec((H, H), lambda i, j: (0, 0)),
        ],
        out_specs=pl.BlockSpec((TM, H), lambda i, j: (i * NT_HALF + j, 0)),
        scratch_shapes=[pltpu.VMEM((TM, H), jnp.float32)],
        compiler_params=pltpu.CompilerParams(
            dimension_semantics=("parallel", "arbitrary"),
            vmem_limit_bytes=50 * 1024 * 1024,
        ),
        name="ffn_keys",
    )(seq_flat, embed, W1, b1, W2, b2, gamma, beta, Wk)


BH = B // 2
T = 64
S = 8
NC = L // T
NS = T // S


def _scan_kernel(k_ref, read_ref, N_ref, U_ref, P_ref, A_ref, KN_ref, vpp_ref):
    c = pl.program_id(1)

    @pl.when(c == 0)
    def _():
        N_ref[...] = jnp.zeros_like(N_ref)
        vpp_ref[...] = jnp.zeros_like(vpp_ref)

    k = k_ref[...]
    n2 = jnp.sum(k * k, axis=-1)
    nrm = jnp.sqrt(n2)
    cn = jnp.maximum(nrm, 1e-12)
    KN_ref[...] = k / cn[:, :, None]
    th2 = VT2 * jnp.maximum(n2, 1e-12)

    for g in range(BH):
        kng = KN_ref[g]
        P_ref[g] = jnp.dot(kng, N_ref[g], preferred_element_type=jnp.float32)
        A_ref[g] = jax.lax.dot_general(
            kng, kng, (((1,), (1,)), ((), ())),
            preferred_element_type=jnp.float32)

    vpp = vpp_ref[...]
    vp = vpp
    for sj in range(NS):
        s0 = sj * S
        if sj > 0:
            for g in range(BH):
                ablk = A_ref[g, s0:s0 + S, :][:, 0:s0]
                P_ref[g, s0:s0 + S, :] = P_ref[g, s0:s0 + S, :] + jnp.dot(
                    ablk, U_ref[g, 0:s0, :], preferred_element_type=jnp.float32)
        base = P_ref[:, s0:s0 + S, :]
        pend = [base[:, ii, :] for ii in range(S)]
        for si in range(S):
            t = s0 + si
            vp = pend[si]
            d = vp - vpp
            d2 = jnp.sum(d * d, axis=-1, keepdims=True)
            valid = (c * T + t) <= (L - 2)
            fire = jnp.logical_and(d2 >= th2[:, t:t + 1], valid)
            w = jnp.where(fire, EMA, 0.0)
            u = w * (k_ref[:, t, :] - vp)
            U_ref[:, t, :] = u
            if si + 1 < S:
                arow = A_ref[:, t, :]
                for ii in range(si + 1, S):
                    coeff = arow[:, s0 + ii:s0 + ii + 1]
                    pend[ii] = pend[ii] + coeff * u
            vpp = vp
    vpp_ref[...] = vp

    @pl.when(c != NC - 1)
    def _():
        for g in range(BH):
            N_ref[g] = N_ref[g] + jax.lax.dot_general(
                KN_ref[g], U_ref[g], (((0,), (0,)), ((), ())),
                preferred_element_type=jnp.float32)

    @pl.when(c == NC - 1)
    def _():
        read_ref[...] = vp * cn[:, T - 1:T]


def _scan_call(k_all):
    return pl.pallas_call(
        _scan_kernel,
        out_shape=jax.ShapeDtypeStruct((B, H), jnp.float32),
        grid=(2, NC),
        in_specs=[pl.BlockSpec((BH, T, H), lambda i, c: (i, c, 0))],
        out_specs=pl.BlockSpec((BH, H), lambda i, c: (i, 0)),
        scratch_shapes=[
            pltpu.VMEM((BH, H, H), jnp.float32),
            pltpu.VMEM((BH, T, H), jnp.float32),
            pltpu.VMEM((BH, T, H), jnp.float32),
            pltpu.VMEM((BH, T, T), jnp.float32),
            pltpu.VMEM((BH, T, H), jnp.float32),
            pltpu.VMEM((BH, H), jnp.float32),
        ],
        compiler_params=pltpu.CompilerParams(
            dimension_semantics=("parallel", "arbitrary"),
            vmem_limit_bytes=50 * 1024 * 1024,
        ),
        name="velocity_gate_scan",
    )(k_all)


TV = 3200
NV = V // TV
NV_HALF = NV // 2


def _out_kernel(read_ref, Wr_ref, br_ref, Wo_ref, bo_ref, out_ref):
    t0 = jnp.dot(read_ref[...], Wr_ref[...],
                 preferred_element_type=jnp.float32) + br_ref[...]
    out_ref[...] = jnp.dot(t0, Wo_ref[...],
                           preferred_element_type=jnp.float32) + bo_ref[...]


def _out_call(read, Wr, br, Wo, bo):
    return pl.pallas_call(
        _out_kernel,
        out_shape=jax.ShapeDtypeStruct((B, V), jnp.float32),
        grid=(2, NV_HALF),
        in_specs=[
            pl.BlockSpec((B, H), lambda i, j: (0, 0)),
            pl.BlockSpec((H, H), lambda i, j: (0, 0)),
            pl.BlockSpec((1, H), lambda i, j: (0, 0)),
            pl.BlockSpec((H, TV), lambda i, j: (0, i * NV_HALF + j)),
            pl.BlockSpec((1, TV), lambda i, j: (0, i * NV_HALF + j)),
        ],
        out_specs=pl.BlockSpec((B, TV), lambda i, j: (0, i * NV_HALF + j)),
        compiler_params=pltpu.CompilerParams(
            dimension_semantics=("parallel", "arbitrary"),
            vmem_limit_bytes=50 * 1024 * 1024,
        ),
        name="out_proj",
    )(read, Wr, br, Wo, bo)


def kernel(seq, embed, W1, b1, W2, b2, gamma, beta, Wk, Wr, br, Wo, bo):
    seq_flat = seq.reshape(-1)
    k_flat = _ffn_call(seq_flat, embed, W1, b1.reshape(1, -1), W2,
                       b2.reshape(1, -1), gamma.reshape(1, -1),
                       beta.reshape(1, -1), Wk)
    k_all = k_flat.reshape(B, L, H)
    read = _scan_call(k_all)
    return _out_call(read, Wr, br.reshape(1, -1), Wo, bo.reshape(1, -1))
